# baseline (speedup 1.0000x reference)
import functools

import jax
import jax.numpy as jnp
from jax import lax
from jax.experimental import pallas as pl
from jax.experimental.pallas import tpu as pltpu
from jax.experimental.pallas import tpu_sc as plsc

D = 768
NJ = D // 16
CH = 32
NCH = 8
G = 8
NB = 4
S = 2048
F32 = jnp.float32


def _rsqrt16(x):
    i = plsc.bitcast(x, jnp.int32)
    i = jnp.int32(0x5F3759DF) - lax.shift_right_logical(i, 1)
    y = plsc.bitcast(i, F32)
    for _ in range(3):
        y = y * (1.5 - 0.5 * x * y * y)
    return y


def _sc_body(ids_hbm, sids_hbm, tok_hbm, pos_hbm, seg_hbm, gam_hbm, bet_hbm,
             out_hbm,
             tok_v, pp_v, seg_v, segd_v, gam_v, bet_v, ids_v, sids_v,
             sem_g, sem_s):
    c = lax.axis_index("c")
    s = lax.axis_index("s")
    wid = s * 2 + c
    pbase = wid * 64

    pltpu.async_copy(ids_hbm.at[0, pl.ds(pbase, 2 * CH)], ids_v.at[0], sem_s)
    pltpu.make_async_copy(
        ids_hbm.at[0, pl.ds(pbase, 2 * CH)], ids_v.at[0], sem_s).wait()
    pltpu.async_copy(
        tok_hbm.at[ids_v.at[0, pl.ds(0, CH)]], tok_v.at[pl.ds(0, CH)], sem_g
    )
    for bb in range(1, NB):
        pltpu.async_copy(ids_hbm.at[bb, pl.ds(pbase, 2 * CH)], ids_v.at[bb], sem_s)
    for bb in range(NB):
        pltpu.async_copy(
            sids_hbm.at[bb, pl.ds(pbase, 2 * CH)],
            sids_v.at[pl.ds(bb * 2 * CH, 2 * CH)], sem_s)
    pltpu.async_copy(seg_hbm, seg_v, sem_s)
    pltpu.async_copy(gam_hbm, gam_v, sem_s)
    pltpu.async_copy(bet_hbm, bet_v, sem_s)
    pltpu.async_copy(pos_hbm.at[pl.ds(pbase, 2 * CH)], pp_v, sem_s)
    for bb in range(1, NB):
        pltpu.make_async_copy(
            ids_hbm.at[bb, pl.ds(pbase, 2 * CH)], ids_v.at[bb], sem_s).wait()
    for bb in range(NB):
        pltpu.make_async_copy(
            sids_hbm.at[bb, pl.ds(pbase, 2 * CH)],
            sids_v.at[pl.ds(bb * 2 * CH, 2 * CH)], sem_s).wait()
    pltpu.make_async_copy(seg_hbm, seg_v, sem_s).wait()
    pltpu.make_async_copy(gam_hbm, gam_v, sem_s).wait()
    pltpu.make_async_copy(bet_hbm, bet_v, sem_s).wait()
    pltpu.make_async_copy(pos_hbm.at[pl.ds(pbase, 2 * CH)], pp_v, sem_s).wait()

    for j in range(NJ):
        sl = pl.ds(j * 16, 16)
        segd_v[sl] = seg_v[1, sl] - seg_v[0, sl]

    @plsc.parallel_loop(0, 2 * CH, unroll=4)
    def _(t):
        for j in range(NJ):
            sl = pl.ds(j * 16, 16)
            pp_v[t, sl] = pp_v[t, sl] + seg_v[0, sl]

    def chunk_body(k, carry):
        slot = lax.rem(k, 2)
        sl0 = slot * CH
        o0 = (1 - slot) * CH
        b = lax.rem(k, NB)
        h = lax.div(k, NB)
        tb = b * S + pbase + h * CH

        @pl.when(k >= 1)
        def _():
            km = k - 1
            pbm = pbase + lax.div(km, NB) * CH
            pltpu.make_async_copy(
                tok_v.at[pl.ds(o0, CH)],
                out_hbm.at[lax.rem(km, NB), pl.ds(pbm, CH)], sem_s
            ).wait()

        pltpu.make_async_copy(
            tok_hbm.at[ids_v.at[b, pl.ds(h * CH, CH)]],
            tok_v.at[pl.ds(sl0, CH)], sem_g
        ).wait()

        @pl.when(k <= NCH - 2)
        def _():
            bn = lax.rem(k + 1, NB)
            hn = lax.div(k + 1, NB)
            pltpu.async_copy(
                tok_hbm.at[ids_v.at[bn, pl.ds(hn * CH, CH)]],
                tok_v.at[pl.ds(o0, CH)], sem_g
            )

        def group_body(g, carry):
            t0 = sl0 + g * G
            p0 = h * CH + g * G
            so = b * 2 * CH + h * CH + g * G
            sidb = []
            for i in range(G):
                sv = sids_v[pl.ds(so + i, 16)]
                sidb.append(jnp.full((16,), sv[0], jnp.int32).astype(F32))
            zero = jnp.zeros((16,), F32)

            @plsc.parallel_loop(0, NJ, unroll=2, carry=(zero,) * (2 * G))
            def accs(j, c):
                sl = pl.ds(j * 16, 16)
                sd = segd_v[sl]
                out = []
                for i in range(G):
                    v = tok_v[t0 + i, sl] + pp_v[p0 + i, sl] + sidb[i] * sd
                    tok_v[t0 + i, sl] = v
                    out.append((c[2 * i] + v, c[2 * i + 1] + v * v))
                return tuple(x for pair in out for x in pair)

            mb, rs = [], []
            for i in range(G):
                mean = jnp.sum(accs[2 * i]) * (1.0 / D)
                var = jnp.sum(accs[2 * i + 1]) * (1.0 / D) - mean * mean
                rs.append(_rsqrt16(jnp.full((16,), var + 1e-5, F32)))
                mb.append(jnp.full((16,), mean, F32))

            @plsc.parallel_loop(0, NJ, unroll=2)
            def _(j):
                sl = pl.ds(j * 16, 16)
                gj = gam_v[sl]
                bj = bet_v[sl]
                for i in range(G):
                    v = tok_v[t0 + i, sl]
                    tok_v[t0 + i, sl] = (v - mb[i]) * rs[i] * gj + bj
            return carry

        lax.fori_loop(0, CH // G, group_body, 0)

        pltpu.async_copy(
            tok_v.at[pl.ds(sl0, CH)], out_hbm.at[b, pl.ds(pbase + h * CH, CH)],
            sem_s
        )
        return carry

    lax.fori_loop(0, NCH, chunk_body, 0)

    pltpu.make_async_copy(
        tok_v.at[pl.ds(CH, CH)],
        out_hbm.at[NB - 1, pl.ds(pbase + CH, CH)], sem_s
    ).wait()


@jax.jit
def _sc_call(ids, sids, token_table, pos_table, seg_table, ln_gamma, ln_beta):
    mesh = plsc.VectorSubcoreMesh(core_axis_name="c", subcore_axis_name="s")
    run = functools.partial(
        pl.kernel,
        mesh=mesh,
        compiler_params=pltpu.CompilerParams(needs_layout_passes=False, skip_device_barrier=True),
        out_type=jax.ShapeDtypeStruct((NB, S, D), F32),
        scratch_types=[
            pltpu.VMEM((2 * CH, D), F32),
            pltpu.VMEM((2 * CH, D), F32),
            pltpu.VMEM((2, D), F32),
            pltpu.VMEM((D,), F32),
            pltpu.VMEM((D,), F32),
            pltpu.VMEM((D,), F32),
            pltpu.VMEM((NB, 2 * CH), jnp.int32),
            pltpu.VMEM((NB * 2 * CH + 16,), jnp.int32),
            pltpu.SemaphoreType.DMA,
            pltpu.SemaphoreType.DMA,
        ],
    )(_sc_body)
    return run(ids, sids, token_table, pos_table, seg_table, ln_gamma, ln_beta)


def kernel(input_ids, segment_ids, token_table, pos_table, seg_table, ln_gamma, ln_beta):
    return _sc_call(input_ids, segment_ids,
                    token_table, pos_table, seg_table, ln_gamma, ln_beta)


# speedup vs baseline: 1.0050x; 1.0050x over previous
import functools

import jax
import jax.numpy as jnp
from jax import lax
from jax.experimental import pallas as pl
from jax.experimental.pallas import tpu as pltpu
from jax.experimental.pallas import tpu_sc as plsc

D = 768
NJ = D // 16
CH = 32
NCH = 8
G = 8
NB = 4
S = 2048
F32 = jnp.float32


def _rsqrt16(x):
    i = plsc.bitcast(x, jnp.int32)
    i = jnp.int32(0x5F3759DF) - lax.shift_right_logical(i, 1)
    y = plsc.bitcast(i, F32)
    for _ in range(3):
        y = y * (1.5 - 0.5 * x * y * y)
    return y


def _sc_body(ids_hbm, sids_hbm, tok_hbm, pos_hbm, seg_hbm, gam_hbm, bet_hbm,
             out_hbm,
             tok_v, pp_v, seg_v, segd_v, gam_v, bet_v, ids_v, sids_v,
             sem_g, sem_s):
    c = lax.axis_index("c")
    s = lax.axis_index("s")
    wid = s * 2 + c
    pbase = wid * 64

    pltpu.async_copy(ids_hbm.at[0, pl.ds(pbase, 2 * CH)], ids_v.at[0], sem_s)
    pltpu.make_async_copy(
        ids_hbm.at[0, pl.ds(pbase, 2 * CH)], ids_v.at[0], sem_s).wait()
    pltpu.async_copy(
        tok_hbm.at[ids_v.at[0, pl.ds(0, CH)]], tok_v.at[pl.ds(0, CH)], sem_g
    )
    for bb in range(1, NB):
        pltpu.async_copy(ids_hbm.at[bb, pl.ds(pbase, 2 * CH)], ids_v.at[bb], sem_s)
    for bb in range(NB):
        pltpu.async_copy(
            sids_hbm.at[bb, pl.ds(pbase, 2 * CH)],
            sids_v.at[pl.ds(bb * 2 * CH, 2 * CH)], sem_s)
    pltpu.async_copy(seg_hbm, seg_v, sem_s)
    pltpu.async_copy(gam_hbm, gam_v, sem_s)
    pltpu.async_copy(bet_hbm, bet_v, sem_s)
    pltpu.async_copy(pos_hbm.at[pl.ds(pbase, 2 * CH)], pp_v, sem_s)
    for bb in range(1, NB):
        pltpu.make_async_copy(
            ids_hbm.at[bb, pl.ds(pbase, 2 * CH)], ids_v.at[bb], sem_s).wait()
    for bb in range(NB):
        pltpu.make_async_copy(
            sids_hbm.at[bb, pl.ds(pbase, 2 * CH)],
            sids_v.at[pl.ds(bb * 2 * CH, 2 * CH)], sem_s).wait()
    pltpu.make_async_copy(seg_hbm, seg_v, sem_s).wait()
    pltpu.make_async_copy(gam_hbm, gam_v, sem_s).wait()
    pltpu.make_async_copy(bet_hbm, bet_v, sem_s).wait()
    pltpu.make_async_copy(pos_hbm.at[pl.ds(pbase, 2 * CH)], pp_v, sem_s).wait()

    for j in range(NJ):
        sl = pl.ds(j * 16, 16)
        segd_v[sl] = seg_v[1, sl] - seg_v[0, sl]

    @plsc.parallel_loop(0, 2 * CH, unroll=4)
    def _(t):
        for j in range(NJ):
            sl = pl.ds(j * 16, 16)
            pp_v[t, sl] = pp_v[t, sl] + seg_v[0, sl]

    def chunk_body(k, carry):
        slot = lax.rem(k, 2)
        sl0 = slot * CH
        o0 = (1 - slot) * CH
        b = lax.rem(k, NB)
        h = lax.div(k, NB)
        tb = b * S + pbase + h * CH

        @pl.when(k >= 1)
        def _():
            km = k - 1
            pbm = pbase + lax.div(km, NB) * CH
            pltpu.make_async_copy(
                tok_v.at[pl.ds(o0, CH)],
                out_hbm.at[lax.rem(km, NB), pl.ds(pbm, CH)], sem_s
            ).wait()

        pltpu.make_async_copy(
            tok_hbm.at[ids_v.at[b, pl.ds(h * CH, CH)]],
            tok_v.at[pl.ds(sl0, CH)], sem_g
        ).wait()

        @pl.when(k <= NCH - 2)
        def _():
            bn = lax.rem(k + 1, NB)
            hn = lax.div(k + 1, NB)
            pltpu.async_copy(
                tok_hbm.at[ids_v.at[bn, pl.ds(hn * CH, CH)]],
                tok_v.at[pl.ds(o0, CH)], sem_g
            )

        def group_body(g, carry):
            t0 = sl0 + g * G
            p0 = h * CH + g * G
            so = b * 2 * CH + h * CH + g * G
            sidb = []
            for i in range(G):
                sv = sids_v[pl.ds(so + i, 16)]
                sidb.append(jnp.full((16,), sv[0], jnp.int32).astype(F32))
            zero = jnp.zeros((16,), F32)

            @plsc.parallel_loop(0, NJ, unroll=2, carry=(zero,) * (2 * G))
            def accs(j, c):
                sl = pl.ds(j * 16, 16)
                sd = segd_v[sl]
                out = []
                for i in range(G):
                    v = tok_v[t0 + i, sl] + pp_v[p0 + i, sl] + sidb[i] * sd
                    tok_v[t0 + i, sl] = v
                    out.append((c[2 * i] + v, c[2 * i + 1] + v * v))
                return tuple(x for pair in out for x in pair)

            mb, rs = [], []
            for i in range(G):
                mean = jnp.sum(accs[2 * i]) * (1.0 / D)
                var = jnp.sum(accs[2 * i + 1]) * (1.0 / D) - mean * mean
                rs.append(_rsqrt16(jnp.full((16,), var + 1e-5, F32)))
                mb.append(jnp.full((16,), mean, F32))

            @plsc.parallel_loop(0, NJ, unroll=2)
            def _(j):
                sl = pl.ds(j * 16, 16)
                gj = gam_v[sl]
                bj = bet_v[sl]
                for i in range(G):
                    v = tok_v[t0 + i, sl]
                    tok_v[t0 + i, sl] = (v - mb[i]) * rs[i] * gj + bj
            return carry

        lax.fori_loop(0, CH // G, group_body, 0)

        pltpu.async_copy(
            tok_v.at[pl.ds(sl0, CH)], out_hbm.at[b, pl.ds(pbase + h * CH, CH)],
            sem_s
        )
        return carry

    lax.fori_loop(0, NCH, chunk_body, 0)

    pltpu.make_async_copy(
        tok_v.at[pl.ds(CH, CH)],
        out_hbm.at[NB - 1, pl.ds(pbase + CH, CH)], sem_s
    ).wait()


@jax.jit
def _sc_call(ids, sids, token_table, pos_table, seg_table, ln_gamma, ln_beta):
    mesh = plsc.VectorSubcoreMesh(core_axis_name="c", subcore_axis_name="s")
    run = functools.partial(
        pl.kernel,
        mesh=mesh,
        compiler_params=pltpu.CompilerParams(needs_layout_passes=False),
        out_type=jax.ShapeDtypeStruct((NB, S, D), F32),
        scratch_types=[
            pltpu.VMEM((2 * CH, D), F32),
            pltpu.VMEM((2 * CH, D), F32),
            pltpu.VMEM((2, D), F32),
            pltpu.VMEM((D,), F32),
            pltpu.VMEM((D,), F32),
            pltpu.VMEM((D,), F32),
            pltpu.VMEM((NB, 2 * CH), jnp.int32),
            pltpu.VMEM((NB * 2 * CH + 16,), jnp.int32),
            pltpu.SemaphoreType.DMA,
            pltpu.SemaphoreType.DMA,
        ],
    )(_sc_body)
    return run(ids, sids, token_table, pos_table, seg_table, ln_gamma, ln_beta)


def kernel(input_ids, segment_ids, token_table, pos_table, seg_table, ln_gamma, ln_beta):
    return _sc_call(input_ids, segment_ids,
                    token_table, pos_table, seg_table, ln_gamma, ln_beta)
